# baseline (speedup 1.0000x reference)
import functools

import jax
import jax.numpy as jnp
from jax import lax
from jax.experimental import pallas as pl
from jax.experimental.pallas import tpu as pltpu
from jax.experimental.pallas import tpu_sc as plsc

_VOCAB = 1000
_D = 1000
_BATCH = 4096
_SEQ = 20
_NW = 32
_CW = 32
_CS = 33
_NPASS = 32
_NT = _CW // 8
_NBLK = 4
_CBT = 125
_IR = 9

_mesh = plsc.VectorSubcoreMesh(core_axis_name="c", subcore_axis_name="s")


@functools.partial(
    pl.kernel,
    mesh=_mesh,
    out_type=jax.ShapeDtypeStruct((_SEQ, _CBT, _NW, 8, 128), jnp.float32),
    scratch_types=[
        pltpu.VMEM((_SEQ, 128), jnp.int32),
        pltpu.VMEM((_SEQ, 64, 16), jnp.int32),
        [pltpu.VMEM((_VOCAB, _CS), jnp.float32) for _ in range(2)],
        [pltpu.VMEM((_NT, 8, 129), jnp.float32) for _ in range(_NBLK)],
        [pltpu.SemaphoreType.DMA for _ in range(2)],
        [pltpu.SemaphoreType.DMA for _ in range(_NBLK)],
        pltpu.SemaphoreType.DMA,
    ],
    compiler_params=pltpu.CompilerParams(
        use_tc_tiling_on_sc=False, needs_layout_passes=False
    ),
)
def _emb_t(xt_hbm, tpad_hbm, out_hbm, idx_v, pairbuf, chunks, blks, csem,
           osem, isem):
    w = lax.axis_index("s") * 2 + lax.axis_index("c")

    iota = lax.iota(jnp.int32, 16)
    k07 = lax.rem(iota, 8)
    bsel8 = (iota // 8) * 8

    pltpu.async_copy(xt_hbm.at[:, pl.ds(w * 128, 128)], idx_v, isem).wait()

    @pl.loop(0, _SEQ)
    def _rep(s):
        sv = jnp.full((16,), s, jnp.int32)
        for g in range(8):
            for j in range(8):
                bv = jnp.full((16,), 16 * g + j, jnp.int32) + bsel8
                iv = plsc.load_gather(idx_v, [sv, bv])
                pairbuf[s, 8 * g + j] = iv

    def load_chunk(p, par):
        off = jnp.minimum(p, _NPASS - 1) * _CW
        pltpu.async_copy(
            tpad_hbm.at[:, pl.ds(off, _CW)],
            chunks[par].at[:, pl.ds(0, _CW)],
            csem[par],
        )

    def wait_chunk(par):
        pltpu.make_async_copy(
            tpad_hbm.at[:, pl.ds(0, _CW)],
            chunks[par].at[:, pl.ds(0, _CW)],
            csem[par],
        ).wait()

    def fill_tiles(par, b, s):
        @plsc.parallel_loop(0, 64, unroll=8)
        def _bp(p_):
            b0 = (p_ // 8) * 16 + lax.rem(p_, 8)
            bv = jnp.full((16,), b0, jnp.int32) + bsel8
            iv = pairbuf[s, p_]
            for t in range(_NT):
                cv = k07 + (8 * t)
                v = plsc.load_gather(chunks[par], [iv, cv])
                tv = jnp.full((16,), t, jnp.int32)
                plsc.store_scatter(blks[b], [tv, k07, bv], v)

    def blk_wait(b, nt):
        pltpu.make_async_copy(
            blks[b].at[pl.ds(0, nt), :, pl.ds(0, 128)],
            out_hbm.at[0, pl.ds(0, nt), 0],
            osem[b],
        ).wait()

    def squad_loop(par, p):
        @pl.loop(0, _SEQ, step=_NBLK)
        def _squad(s0):
            for b in range(_NBLK):
                s = s0 + b
                skip = jnp.logical_and(p == 0, s0 == 0)
                last_rep = jnp.logical_and(p == _NPASS - 1, s0 > 0)

                @pl.when(jnp.logical_and(jnp.logical_not(skip),
                                         jnp.logical_not(last_rep)))
                def _():
                    blk_wait(b, _NT)

                @pl.when(last_rep)
                def _():
                    blk_wait(b, 1)

                fill_tiles(par, b, s)

                @pl.when(p < _NPASS - 1)
                def _():
                    pltpu.async_copy(
                        blks[b].at[:, :, pl.ds(0, 128)],
                        out_hbm.at[s, pl.ds(p * _NT, _NT), w],
                        osem[b],
                    )

                @pl.when(p == _NPASS - 1)
                def _():
                    pltpu.async_copy(
                        blks[b].at[pl.ds(0, 1), :, pl.ds(0, 128)],
                        out_hbm.at[s, pl.ds(_CBT - 1, 1), w],
                        osem[b],
                    )

    load_chunk(0, 0)
    load_chunk(1, 1)

    @pl.loop(0, _NPASS, step=2)
    def _pair(p):
        wait_chunk(0)
        squad_loop(0, p)
        load_chunk(p + 2, 0)
        wait_chunk(1)
        squad_loop(1, p + 1)
        load_chunk(p + 3, 1)

    for b in range(_NBLK):
        blk_wait(b, 1)
    wait_chunk(0)
    wait_chunk(1)


def kernel(x, table):
    xt = x.T.astype(jnp.int32)
    tpad = jnp.pad(table, ((0, 0), (0, 24)))
    out5 = _emb_t(xt, tpad)
    return out5.transpose(2, 4, 0, 1, 3).reshape(_BATCH, _SEQ, _D)


# speedup vs baseline: 1.1497x; 1.1497x over previous
import functools

import jax
import jax.numpy as jnp
from jax import lax
from jax.experimental import pallas as pl
from jax.experimental.pallas import tpu as pltpu
from jax.experimental.pallas import tpu_sc as plsc

_VOCAB = 1000
_D = 1000
_BATCH = 4096
_SEQ = 20
_NW = 32
_CW = 32
_CS = 33
_NPASS = 32
_NT = _CW // 8
_NBLK = 4
_CBT = 125
_IR = 9

_mesh = plsc.VectorSubcoreMesh(core_axis_name="c", subcore_axis_name="s")


@functools.partial(
    pl.kernel,
    mesh=_mesh,
    out_type=jax.ShapeDtypeStruct((_SEQ, _CBT, _NW, 8, 128), jnp.float32),
    scratch_types=[
        pltpu.VMEM((_SEQ, 128), jnp.int32),
        pltpu.VMEM((_SEQ, 64, 16), jnp.int32),
        [pltpu.VMEM((_VOCAB, _CS), jnp.float32) for _ in range(2)],
        [pltpu.VMEM((_NT, 8, 129), jnp.float32) for _ in range(_NBLK)],
        [pltpu.SemaphoreType.DMA for _ in range(2)],
        [pltpu.SemaphoreType.DMA for _ in range(_NBLK)],
        pltpu.SemaphoreType.DMA,
    ],
    compiler_params=pltpu.CompilerParams(
        use_tc_tiling_on_sc=False, needs_layout_passes=False
    ),
)
def _emb_t(xt_hbm, tpad_hbm, out_hbm, idx_v, pairbuf, chunks, blks, csem,
           osem, isem):
    w = lax.axis_index("s") * 2 + lax.axis_index("c")

    iota = lax.iota(jnp.int32, 16)
    k07 = lax.rem(iota, 8)
    bsel8 = (iota // 8) * 8

    pltpu.async_copy(xt_hbm.at[:, pl.ds(w * 128, 128)], idx_v, isem).wait()

    @pl.loop(0, _SEQ)
    def _rep(s):
        sv = jnp.full((16,), s, jnp.int32)
        for g in range(8):
            for j in range(8):
                bv = jnp.full((16,), 16 * g + j, jnp.int32) + bsel8
                iv = plsc.load_gather(idx_v, [sv, bv])
                pairbuf[s, 8 * g + j] = iv * _CS

    def load_chunk(p, par):
        off = jnp.minimum(p, _NPASS - 1) * _CW
        pltpu.async_copy(
            tpad_hbm.at[:, pl.ds(off, _CW)],
            chunks[par].at[:, pl.ds(0, _CW)],
            csem[par],
        )

    def wait_chunk(par):
        pltpu.make_async_copy(
            tpad_hbm.at[:, pl.ds(0, _CW)],
            chunks[par].at[:, pl.ds(0, _CW)],
            csem[par],
        ).wait()

    def fill_tiles(par, b, s):
        z16 = jnp.zeros((16,), jnp.int32)

        @plsc.parallel_loop(0, 64, unroll=4)
        def _bp(p_):
            b0 = (p_ // 8) * 16 + lax.rem(p_, 8)
            bv = jnp.full((16,), b0, jnp.int32) + bsel8
            iv33 = pairbuf[s, p_]
            for t in range(_NT):
                cv = k07 + (8 * t)
                v = plsc.load_gather(chunks[par], [z16, iv33 + cv])
                tv = jnp.full((16,), t, jnp.int32)
                plsc.store_scatter(blks[b], [tv, k07, bv], v)

    def blk_wait(b, nt):
        pltpu.make_async_copy(
            blks[b].at[pl.ds(0, nt), :, pl.ds(0, 128)],
            out_hbm.at[0, pl.ds(0, nt), 0],
            osem[b],
        ).wait()

    def squad_loop(par, p):
        @pl.loop(0, _SEQ, step=_NBLK)
        def _squad(s0):
            for b in range(_NBLK):
                s = s0 + b
                skip = jnp.logical_and(p == 0, s0 == 0)
                last_rep = jnp.logical_and(p == _NPASS - 1, s0 > 0)

                @pl.when(jnp.logical_and(jnp.logical_not(skip),
                                         jnp.logical_not(last_rep)))
                def _():
                    blk_wait(b, _NT)

                @pl.when(last_rep)
                def _():
                    blk_wait(b, 1)

                fill_tiles(par, b, s)

                @pl.when(p < _NPASS - 1)
                def _():
                    pltpu.async_copy(
                        blks[b].at[:, :, pl.ds(0, 128)],
                        out_hbm.at[s, pl.ds(p * _NT, _NT), w],
                        osem[b],
                    )

                @pl.when(p == _NPASS - 1)
                def _():
                    pltpu.async_copy(
                        blks[b].at[pl.ds(0, 1), :, pl.ds(0, 128)],
                        out_hbm.at[s, pl.ds(_CBT - 1, 1), w],
                        osem[b],
                    )

    load_chunk(0, 0)
    load_chunk(1, 1)

    @pl.loop(0, _NPASS, step=2)
    def _pair(p):
        wait_chunk(0)
        squad_loop(0, p)
        load_chunk(p + 2, 0)
        wait_chunk(1)
        squad_loop(1, p + 1)
        load_chunk(p + 3, 1)

    for b in range(_NBLK):
        blk_wait(b, 1)
    wait_chunk(0)
    wait_chunk(1)


def kernel(x, table):
    xt = x.T.astype(jnp.int32)
    tpad = jnp.pad(table, ((0, 0), (0, 24)))
    out5 = _emb_t(xt, tpad)
    return out5.transpose(2, 4, 0, 1, 3).reshape(_BATCH, _SEQ, _D)


# speedup vs baseline: 1.1783x; 1.0249x over previous
import functools

import jax
import jax.numpy as jnp
from jax import lax
from jax.experimental import pallas as pl
from jax.experimental.pallas import tpu as pltpu
from jax.experimental.pallas import tpu_sc as plsc

_VOCAB = 1000
_D = 1000
_BATCH = 4096
_SEQ = 20
_NW = 32
_CW = 32
_CS = 33
_NPASS = 32
_NT = _CW // 8
_NBLK = 4
_CBT = 125
_IR = 9

_mesh = plsc.VectorSubcoreMesh(core_axis_name="c", subcore_axis_name="s")


@functools.partial(
    pl.kernel,
    mesh=_mesh,
    out_type=jax.ShapeDtypeStruct((_SEQ, _CBT, _NW, 8, 128), jnp.float32),
    scratch_types=[
        pltpu.VMEM((_SEQ, 128), jnp.int32),
        pltpu.VMEM((_SEQ, 64, 16), jnp.int32),
        [pltpu.VMEM((_VOCAB, _CS), jnp.float32) for _ in range(2)],
        [pltpu.VMEM((_NT, 8, 129), jnp.float32) for _ in range(_NBLK)],
        [pltpu.SemaphoreType.DMA for _ in range(2)],
        [pltpu.SemaphoreType.DMA for _ in range(_NBLK)],
        pltpu.SemaphoreType.DMA,
    ],
    compiler_params=pltpu.CompilerParams(
        use_tc_tiling_on_sc=False, needs_layout_passes=False
    ),
)
def _emb_t(xt_hbm, tpad_hbm, out_hbm, idx_v, pairbuf, chunks, blks, csem,
           osem, isem):
    w = lax.axis_index("s") * 2 + lax.axis_index("c")

    iota = lax.iota(jnp.int32, 16)
    k07 = lax.rem(iota, 8)
    bsel8 = (iota // 8) * 8

    pltpu.async_copy(xt_hbm.at[:, pl.ds(w * 128, 128)], idx_v, isem).wait()

    @pl.loop(0, _SEQ)
    def _rep(s):
        sv = jnp.full((16,), s, jnp.int32)
        for g in range(8):
            for j in range(8):
                bv = jnp.full((16,), 16 * g + j, jnp.int32) + bsel8
                iv = plsc.load_gather(idx_v, [sv, bv])
                pairbuf[s, 8 * g + j] = iv

    def load_chunk(p, par):
        off = jnp.minimum(p, _NPASS - 1) * _CW
        pltpu.async_copy(
            tpad_hbm.at[:, pl.ds(off, _CW)],
            chunks[par].at[:, pl.ds(0, _CW)],
            csem[par],
        )

    def wait_chunk(par):
        pltpu.make_async_copy(
            tpad_hbm.at[:, pl.ds(0, _CW)],
            chunks[par].at[:, pl.ds(0, _CW)],
            csem[par],
        ).wait()

    def fill_tiles(par, b, s):
        @plsc.parallel_loop(0, 64, unroll=4)
        def _bp(p_):
            b0 = (p_ // 8) * 16 + lax.rem(p_, 8)
            bv = jnp.full((16,), b0, jnp.int32) + bsel8
            iv = pairbuf[s, p_]
            for t in range(_NT):
                cv = k07 + (8 * t)
                v = plsc.load_gather(chunks[par], [iv, cv])
                tv = jnp.full((16,), t, jnp.int32)
                plsc.store_scatter(blks[b], [tv, k07, bv], v)

    def blk_wait(b, nt):
        pltpu.make_async_copy(
            blks[b].at[pl.ds(0, nt), :, pl.ds(0, 128)],
            out_hbm.at[0, pl.ds(0, nt), 0],
            osem[b],
        ).wait()

    def squad_loop(par, p):
        @pl.loop(0, _SEQ, step=_NBLK)
        def _squad(s0):
            for b in range(_NBLK):
                s = s0 + b
                skip = jnp.logical_and(p == 0, s0 == 0)
                last_rep = jnp.logical_and(p == _NPASS - 1, s0 > 0)

                @pl.when(jnp.logical_and(jnp.logical_not(skip),
                                         jnp.logical_not(last_rep)))
                def _():
                    blk_wait(b, _NT)

                @pl.when(last_rep)
                def _():
                    blk_wait(b, 1)

                fill_tiles(par, b, s)

                @pl.when(p < _NPASS - 1)
                def _():
                    pltpu.async_copy(
                        blks[b].at[:, :, pl.ds(0, 128)],
                        out_hbm.at[s, pl.ds(p * _NT, _NT), w],
                        osem[b],
                    )

                @pl.when(p == _NPASS - 1)
                def _():
                    pltpu.async_copy(
                        blks[b].at[pl.ds(0, 1), :, pl.ds(0, 128)],
                        out_hbm.at[s, pl.ds(_CBT - 1, 1), w],
                        osem[b],
                    )

    load_chunk(0, 0)
    load_chunk(1, 1)

    @pl.loop(0, _NPASS, step=2)
    def _pair(p):
        wait_chunk(0)
        squad_loop(0, p)
        load_chunk(p + 2, 0)
        wait_chunk(1)
        squad_loop(1, p + 1)
        load_chunk(p + 3, 1)

    for b in range(_NBLK):
        blk_wait(b, 1)
    wait_chunk(0)
    wait_chunk(1)


def kernel(x, table):
    xt = x.T.astype(jnp.int32)
    tpad = jnp.pad(table, ((0, 0), (0, 24)))
    out5 = _emb_t(xt, tpad)
    return out5.transpose(2, 4, 0, 1, 3).reshape(_BATCH, _SEQ, _D)
